# baseline (speedup 1.0000x reference)
import functools
import math

import jax
import jax.numpy as jnp
from jax import lax
from jax.experimental import pallas as pl
from jax.experimental.pallas import tpu as pltpu
from jax.experimental.pallas import tpu_sc as plsc

N = 50000
E = 800000
F_IN = 128
H = 64
L = 16
R = 512
C = 40
ALPHA = 0.1
THETA = 0.5
H1 = (R - H) // 3 + H
H2 = 2 * (R - H) // 3 + H
H1P = 256
H2P = 384

NC = 2
NS = 16
NQ = 4
QW = H // NQ
G = 128
KJ = 8
CHUNK = G * KJ
NBUF = 3
CH_PER_TILE = NBUF * (-(-E // (NS * CHUNK * NBUF)))
E_PAD = CH_PER_TILE * NS * CHUNK
EDGE_ROWS = E_PAD // G
ROWS_PER_TILE = EDGE_ROWS // NS
ACC_ROWS = 50048
STRIPE = ACC_ROWS // NS

BN = 1000
NBLK = N // BN


def _sc_scatter_body(src_hbm, dst_hbm, xs_hbm, zeros_hbm, out_hbm,
                     src0, src1, src2, dst0, dst1, dst2,
                     rows0, rows1, rows2, acc,
                     gs0, gs1, gs2, ss0, ss1, ss2):
    c = lax.axis_index("c")
    s = lax.axis_index("s")
    bufs = [(src0, dst0, rows0, gs0, ss0),
            (src1, dst1, rows1, gs1, ss1),
            (src2, dst2, rows2, gs2, ss2)]

    def load_and_gather(k, it, q):
        srcb, dstb, rowsb, gs, _ = bufs[k]
        row = s * ROWS_PER_TILE + it * KJ
        pltpu.sync_copy(src_hbm.at[q, pl.ds(row, KJ)], srcb)
        pltpu.sync_copy(dst_hbm.at[pl.ds(row, KJ)], dstb)
        for j in range(KJ):
            pltpu.async_copy(xs_hbm.at[srcb.at[j]],
                             rowsb.at[pl.ds(j * G, G)], gs)

    def wait_gather(k):
        _, _, rowsb, gs, _ = bufs[k]
        pltpu.make_async_copy(xs_hbm.at[pl.ds(0, CHUNK)], rowsb, gs).wait()

    def scatter(k):
        _, dstb, rowsb, _, ss = bufs[k]
        for j in range(KJ):
            pltpu.async_copy(rowsb.at[pl.ds(j * G, G)],
                             acc.at[dstb.at[j]], ss, add=True)

    def wait_scatter(k):
        _, _, rowsb, _, ss = bufs[k]
        pltpu.make_async_copy(rowsb, acc.at[pl.ds(0, CHUNK)], ss).wait()

    for p in range(NQ // NC):
        q = c * (NQ // NC) + p
        pltpu.sync_copy(zeros_hbm.at[pl.ds(s * STRIPE, STRIPE)],
                        acc.at[pl.ds(s * STRIPE, STRIPE)])
        plsc.subcore_barrier()

        def body(it, carry):
            load_and_gather(0, it, q)
            wait_gather(0)
            scatter(0)
            wait_scatter(0)
            return carry

        lax.fori_loop(0, CH_PER_TILE, body, 0)
        plsc.subcore_barrier()
        pltpu.sync_copy(acc.at[pl.ds(s * STRIPE, STRIPE)],
                        out_hbm.at[q, pl.ds(s * STRIPE, STRIPE)])
        plsc.subcore_barrier()


_sc_scatter = functools.partial(
    pl.kernel,
    mesh=plsc.VectorSubcoreMesh(core_axis_name="c", subcore_axis_name="s"),
    compiler_params=pltpu.CompilerParams(use_tc_tiling_on_sc=False),
    out_type=jax.ShapeDtypeStruct((NQ, ACC_ROWS, QW), jnp.float32),
    scratch_types=(
        [pltpu.VMEM((KJ, G), jnp.int32)] * 6
        + [pltpu.VMEM((CHUNK, QW), jnp.float32)] * 3
        + [pltpu.VMEM_SHARED((ACC_ROWS, QW), jnp.float32)]
        + [pltpu.SemaphoreType.DMA] * 6
    ),
)(_sc_scatter_body)


def _lin0_body(xp_ref, w_ref, b_ref, cnt_ref, x0_ref, dinv_ref, xs_ref):
    deg = cnt_ref[0][:, :1] + 1.0
    d = lax.rsqrt(deg)
    x0 = jnp.maximum(
        jnp.dot(xp_ref[...], w_ref[...], preferred_element_type=jnp.float32)
        + b_ref[...], 0.0)
    x0_ref[...] = x0
    dinv_ref[...] = d
    xs_ref[...] = d * x0


def _lin0(x_param, lin0_w, lin0_b, cnt):
    return pl.pallas_call(
        _lin0_body,
        grid=(NBLK,),
        in_specs=[
            pl.BlockSpec((BN, F_IN), lambda i: (i, 0)),
            pl.BlockSpec((F_IN, H), lambda i: (0, 0)),
            pl.BlockSpec((1, H), lambda i: (0, 0)),
            pl.BlockSpec((NQ, BN, QW), lambda i: (0, i, 0)),
        ],
        out_specs=[
            pl.BlockSpec((BN, H), lambda i: (i, 0)),
            pl.BlockSpec((BN, 1), lambda i: (i, 0)),
            pl.BlockSpec((BN, H), lambda i: (i, 0)),
        ],
        out_shape=[
            jax.ShapeDtypeStruct((N, H), jnp.float32),
            jax.ShapeDtypeStruct((N, 1), jnp.float32),
            jax.ShapeDtypeStruct((N, H), jnp.float32),
        ],
    )(x_param, lin0_w, lin0_b, cnt)


def _layer_body(agg_ref, x_ref, x0_ref, dinv_ref, w_ref, xp_ref, xs_ref):
    aggc = jnp.concatenate([agg_ref[i] for i in range(NQ)], axis=1)
    d = dinv_ref[...]
    h = (1.0 - ALPHA) * (d * aggc + (d * d) * x_ref[...]) + ALPHA * x0_ref[...]
    xp = jnp.maximum(
        jnp.dot(h, w_ref[...], preferred_element_type=jnp.float32), 0.0)
    xp_ref[...] = xp
    xs_ref[...] = d * xp


def _layer(agg, x, x0, dinv, w):
    return pl.pallas_call(
        _layer_body,
        grid=(NBLK,),
        in_specs=[
            pl.BlockSpec((NQ, BN, QW), lambda i: (0, i, 0)),
            pl.BlockSpec((BN, H), lambda i: (i, 0)),
            pl.BlockSpec((BN, H), lambda i: (i, 0)),
            pl.BlockSpec((BN, 1), lambda i: (i, 0)),
            pl.BlockSpec((H, H), lambda i: (0, 0)),
        ],
        out_specs=[
            pl.BlockSpec((BN, H), lambda i: (i, 0)),
            pl.BlockSpec((BN, H), lambda i: (i, 0)),
        ],
        out_shape=[
            jax.ShapeDtypeStruct((N, H), jnp.float32),
            jax.ShapeDtypeStruct((N, H), jnp.float32),
        ],
    )(agg, x, x0, dinv, w)


def _mlp_body(x_ref, w1_ref, b1_ref, w2_ref, b2_ref, w3_ref, b3_ref,
              ow_ref, ob_ref, out_ref):
    t = jnp.maximum(
        jnp.dot(x_ref[...], w1_ref[...], preferred_element_type=jnp.float32)
        + b1_ref[...], 0.0)
    t = jnp.maximum(
        jnp.dot(t, w2_ref[...], preferred_element_type=jnp.float32)
        + b2_ref[...], 0.0)
    t = jnp.dot(t, w3_ref[...], preferred_element_type=jnp.float32) + b3_ref[...]
    out_ref[...] = (
        jnp.dot(t, ow_ref[...], preferred_element_type=jnp.float32) + ob_ref[...])


def _mlp(x, w1, b1, w2, b2, w3, b3, ow, ob):
    return pl.pallas_call(
        _mlp_body,
        grid=(NBLK,),
        in_specs=[
            pl.BlockSpec((BN, H), lambda i: (i, 0)),
            pl.BlockSpec((H, H1P), lambda i: (0, 0)),
            pl.BlockSpec((1, H1P), lambda i: (0, 0)),
            pl.BlockSpec((H1P, H2P), lambda i: (0, 0)),
            pl.BlockSpec((1, H2P), lambda i: (0, 0)),
            pl.BlockSpec((H2P, R), lambda i: (0, 0)),
            pl.BlockSpec((1, R), lambda i: (0, 0)),
            pl.BlockSpec((R, C), lambda i: (0, 0)),
            pl.BlockSpec((1, C), lambda i: (0, 0)),
        ],
        out_specs=pl.BlockSpec((BN, C), lambda i: (i, 0)),
        out_shape=jax.ShapeDtypeStruct((N, C), jnp.float32),
    )(x, w1, b1, w2, b2, w3, b3, ow, ob)


def kernel(edge_index, x_param, lin0_w, lin0_b, conv_w, mlp_w1, mlp_b1,
           mlp_w2, mlp_b2, mlp_w3, mlp_b3, out_w, out_b):
    src = edge_index[0]
    dst = edge_index[1]
    pad = E_PAD - E
    srcp = jnp.concatenate([src, jnp.zeros((pad,), jnp.int32)])
    src4 = (NQ * srcp)[None, :] + jnp.arange(NQ, dtype=jnp.int32)[:, None]
    src4 = src4.reshape(NQ, EDGE_ROWS, G)
    dstp = jnp.concatenate([dst, jnp.full((pad,), N, jnp.int32)])
    dst2 = dstp.reshape(EDGE_ROWS, G)
    zeros = jnp.zeros((ACC_ROWS, QW), jnp.float32)
    ones_tbl = jnp.ones((NQ * N, QW), jnp.float32)

    betas = jnp.asarray([math.log(THETA / (i + 1) + 1.0) for i in range(L)],
                        jnp.float32)
    eye = jnp.eye(H, dtype=jnp.float32)
    wp = (1.0 - betas)[:, None, None] * eye + betas[:, None, None] * conv_w

    w1 = jnp.pad(mlp_w1, ((0, 0), (0, H1P - H1)))
    b1 = jnp.pad(mlp_b1, (0, H1P - H1)).reshape(1, H1P)
    w2 = jnp.pad(mlp_w2, ((0, H1P - H1), (0, H2P - H2)))
    b2 = jnp.pad(mlp_b2, (0, H2P - H2)).reshape(1, H2P)
    w3 = jnp.pad(mlp_w3, ((0, H2P - H2), (0, 0)))
    b3 = mlp_b3.reshape(1, R)
    ob = out_b.reshape(1, C)

    cnt = _sc_scatter(src4, dst2, ones_tbl, zeros)
    x0, dinv, xs = _lin0(x_param, lin0_w, lin0_b.reshape(1, H), cnt)
    x = x0
    for i in range(L):
        agg = _sc_scatter(src4, dst2, xs.reshape(NQ * N, QW), zeros)
        x, xs = _layer(agg, x, x0, dinv, wp[i])
    return _mlp(x, w1, b1, w2, b2, w3, b3, out_w, ob)


# speedup vs baseline: 1.9381x; 1.9381x over previous
import functools
import math

import jax
import jax.numpy as jnp
from jax import lax
from jax.experimental import pallas as pl
from jax.experimental.pallas import tpu as pltpu
from jax.experimental.pallas import tpu_sc as plsc

N = 50000
E = 800000
F_IN = 128
H = 64
L = 16
R = 512
C = 40
ALPHA = 0.1
THETA = 0.5
H1 = (R - H) // 3 + H
H2 = 2 * (R - H) // 3 + H
H1P = 256
H2P = 384

NC = 2
NS = 16
NQ = 4
QW = H // NQ
G = 128
KJ = 8
CHUNK = G * KJ
CH_PER_TILE = -(-E // (NS * CHUNK))
E_PAD = CH_PER_TILE * NS * CHUNK
EDGE_ROWS = E_PAD // G
ROWS_PER_TILE = EDGE_ROWS // NS
ACC_ROWS = 50048
STRIPE = ACC_ROWS // NS

BN = 1000
NBLK = N // BN


def _sc_scatter_body(src_hbm, dst_hbm, xs_hbm, zeros_hbm, out_hbm,
                     src_v, dst_v, rows_v, acc, sem):
    c = lax.axis_index("c")
    s = lax.axis_index("s")

    for p in range(NQ // NC):
        q = c * (NQ // NC) + p
        pltpu.sync_copy(zeros_hbm.at[pl.ds(s * STRIPE, STRIPE)],
                        acc.at[pl.ds(s * STRIPE, STRIPE)])
        plsc.subcore_barrier()

        def body(it, carry):
            row = s * ROWS_PER_TILE + it * KJ
            pltpu.sync_copy(src_hbm.at[q, pl.ds(row, KJ)], src_v)
            pltpu.sync_copy(dst_hbm.at[pl.ds(row, KJ)], dst_v)
            copies = [
                pltpu.async_copy(xs_hbm.at[src_v.at[j]],
                                 rows_v.at[pl.ds(j * G, G)], sem)
                for j in range(KJ)
            ]
            for cp in copies:
                cp.wait()
            for j in range(KJ):
                pltpu.sync_copy(rows_v.at[pl.ds(j * G, G)],
                                acc.at[dst_v.at[j]], add=True)
            return carry

        lax.fori_loop(0, CH_PER_TILE, body, 0)
        plsc.subcore_barrier()
        pltpu.sync_copy(acc.at[pl.ds(s * STRIPE, STRIPE)],
                        out_hbm.at[pl.ds(s * STRIPE, STRIPE),
                                   pl.ds(q * QW, QW)])
        plsc.subcore_barrier()


_sc_scatter = functools.partial(
    pl.kernel,
    mesh=plsc.VectorSubcoreMesh(core_axis_name="c", subcore_axis_name="s"),
    compiler_params=pltpu.CompilerParams(use_tc_tiling_on_sc=False),
    out_type=jax.ShapeDtypeStruct((ACC_ROWS, H), jnp.float32),
    scratch_types=[
        pltpu.VMEM((KJ, G), jnp.int32),
        pltpu.VMEM((KJ, G), jnp.int32),
        pltpu.VMEM((CHUNK, QW), jnp.float32),
        pltpu.VMEM_SHARED((ACC_ROWS, QW), jnp.float32),
        pltpu.SemaphoreType.DMA,
    ],
)(_sc_scatter_body)


def _lin0_body(xp_ref, w_ref, b_ref, cnt_ref, x0_ref, dinv_ref, xs_ref):
    deg = cnt_ref[:, :1] + 1.0
    d = lax.rsqrt(deg)
    x0 = jnp.maximum(
        jnp.dot(xp_ref[...], w_ref[...], preferred_element_type=jnp.float32)
        + b_ref[...], 0.0)
    x0_ref[...] = x0
    dinv_ref[...] = d
    xs_ref[...] = d * x0


def _lin0(x_param, lin0_w, lin0_b, cnt):
    return pl.pallas_call(
        _lin0_body,
        grid=(NBLK,),
        in_specs=[
            pl.BlockSpec((BN, F_IN), lambda i: (i, 0)),
            pl.BlockSpec((F_IN, H), lambda i: (0, 0)),
            pl.BlockSpec((1, H), lambda i: (0, 0)),
            pl.BlockSpec((BN, H), lambda i: (i, 0)),
        ],
        out_specs=[
            pl.BlockSpec((BN, H), lambda i: (i, 0)),
            pl.BlockSpec((BN, 1), lambda i: (i, 0)),
            pl.BlockSpec((BN, H), lambda i: (i, 0)),
        ],
        out_shape=[
            jax.ShapeDtypeStruct((N, H), jnp.float32),
            jax.ShapeDtypeStruct((N, 1), jnp.float32),
            jax.ShapeDtypeStruct((N, H), jnp.float32),
        ],
    )(x_param, lin0_w, lin0_b, cnt)


def _layer_body(agg_ref, x_ref, x0_ref, dinv_ref, w_ref, xp_ref, xs_ref):
    aggc = agg_ref[...]
    d = dinv_ref[...]
    h = (1.0 - ALPHA) * (d * aggc + (d * d) * x_ref[...]) + ALPHA * x0_ref[...]
    xp = jnp.maximum(
        jnp.dot(h, w_ref[...], preferred_element_type=jnp.float32), 0.0)
    xp_ref[...] = xp
    xs_ref[...] = d * xp


def _layer(agg, x, x0, dinv, w):
    return pl.pallas_call(
        _layer_body,
        grid=(NBLK,),
        in_specs=[
            pl.BlockSpec((BN, H), lambda i: (i, 0)),
            pl.BlockSpec((BN, H), lambda i: (i, 0)),
            pl.BlockSpec((BN, H), lambda i: (i, 0)),
            pl.BlockSpec((BN, 1), lambda i: (i, 0)),
            pl.BlockSpec((H, H), lambda i: (0, 0)),
        ],
        out_specs=[
            pl.BlockSpec((BN, H), lambda i: (i, 0)),
            pl.BlockSpec((BN, H), lambda i: (i, 0)),
        ],
        out_shape=[
            jax.ShapeDtypeStruct((N, H), jnp.float32),
            jax.ShapeDtypeStruct((N, H), jnp.float32),
        ],
    )(agg, x, x0, dinv, w)


def _mlp_body(x_ref, w1_ref, b1_ref, w2_ref, b2_ref, w3_ref, b3_ref,
              ow_ref, ob_ref, out_ref):
    t = jnp.maximum(
        jnp.dot(x_ref[...], w1_ref[...], preferred_element_type=jnp.float32)
        + b1_ref[...], 0.0)
    t = jnp.maximum(
        jnp.dot(t, w2_ref[...], preferred_element_type=jnp.float32)
        + b2_ref[...], 0.0)
    t = jnp.dot(t, w3_ref[...], preferred_element_type=jnp.float32) + b3_ref[...]
    out_ref[...] = (
        jnp.dot(t, ow_ref[...], preferred_element_type=jnp.float32) + ob_ref[...])


def _mlp(x, w1, b1, w2, b2, w3, b3, ow, ob):
    return pl.pallas_call(
        _mlp_body,
        grid=(NBLK,),
        in_specs=[
            pl.BlockSpec((BN, H), lambda i: (i, 0)),
            pl.BlockSpec((H, H1P), lambda i: (0, 0)),
            pl.BlockSpec((1, H1P), lambda i: (0, 0)),
            pl.BlockSpec((H1P, H2P), lambda i: (0, 0)),
            pl.BlockSpec((1, H2P), lambda i: (0, 0)),
            pl.BlockSpec((H2P, R), lambda i: (0, 0)),
            pl.BlockSpec((1, R), lambda i: (0, 0)),
            pl.BlockSpec((R, C), lambda i: (0, 0)),
            pl.BlockSpec((1, C), lambda i: (0, 0)),
        ],
        out_specs=pl.BlockSpec((BN, C), lambda i: (i, 0)),
        out_shape=jax.ShapeDtypeStruct((N, C), jnp.float32),
    )(x, w1, b1, w2, b2, w3, b3, ow, ob)


def kernel(edge_index, x_param, lin0_w, lin0_b, conv_w, mlp_w1, mlp_b1,
           mlp_w2, mlp_b2, mlp_w3, mlp_b3, out_w, out_b):
    src = edge_index[0]
    dst = edge_index[1]
    pad = E_PAD - E
    srcp = jnp.concatenate([src, jnp.zeros((pad,), jnp.int32)])
    src4 = (NQ * srcp)[None, :] + jnp.arange(NQ, dtype=jnp.int32)[:, None]
    src4 = src4.reshape(NQ, EDGE_ROWS, G)
    dstp = jnp.concatenate([dst, jnp.full((pad,), N, jnp.int32)])
    dst2 = dstp.reshape(EDGE_ROWS, G)
    zeros = jnp.zeros((ACC_ROWS, QW), jnp.float32)
    ones_tbl = jnp.ones((NQ * N, QW), jnp.float32)

    betas = jnp.asarray([math.log(THETA / (i + 1) + 1.0) for i in range(L)],
                        jnp.float32)
    eye = jnp.eye(H, dtype=jnp.float32)
    wp = (1.0 - betas)[:, None, None] * eye + betas[:, None, None] * conv_w

    w1 = jnp.pad(mlp_w1, ((0, 0), (0, H1P - H1)))
    b1 = jnp.pad(mlp_b1, (0, H1P - H1)).reshape(1, H1P)
    w2 = jnp.pad(mlp_w2, ((0, H1P - H1), (0, H2P - H2)))
    b2 = jnp.pad(mlp_b2, (0, H2P - H2)).reshape(1, H2P)
    w3 = jnp.pad(mlp_w3, ((0, H2P - H2), (0, 0)))
    b3 = mlp_b3.reshape(1, R)
    ob = out_b.reshape(1, C)

    cnt = _sc_scatter(src4, dst2, ones_tbl, zeros)
    x0, dinv, xs = _lin0(x_param, lin0_w, lin0_b.reshape(1, H), cnt)
    x = x0
    for i in range(L):
        agg = _sc_scatter(src4, dst2, xs.reshape(NQ * N, QW), zeros)
        x, xs = _layer(agg, x, x0, dinv, wp[i])
    return _mlp(x, w1, b1, w2, b2, w3, b3, out_w, ob)


# speedup vs baseline: 2.6081x; 1.3457x over previous
import functools
import math

import jax
import jax.numpy as jnp
from jax import lax
from jax.experimental import pallas as pl
from jax.experimental.pallas import tpu as pltpu
from jax.experimental.pallas import tpu_sc as plsc

N = 50000
E = 800000
F_IN = 128
H = 64
L = 16
R = 512
C = 40
ALPHA = 0.1
THETA = 0.5
H1 = (R - H) // 3 + H
H2 = 2 * (R - H) // 3 + H
H1P = 256
H2P = 384

NC = 2
NS = 16
NQ = 4
QW = H // NQ
G = 128
KJ = 8
CHUNK = G * KJ
CH_PER_TILE = -(-E // (NS * CHUNK))
E_PAD = CH_PER_TILE * NS * CHUNK
EDGE_ROWS = E_PAD // G
ROWS_PER_TILE = EDGE_ROWS // NS
ACC_ROWS = 50048
STRIPE = ACC_ROWS // NS

BN = 1000
NBLK = N // BN


def _sc_scatter_body(src_hbm, dst_hbm, xs_hbm, zeros_hbm, out_hbm,
                     src0, src1, dst0, dst1, rows0, rows1, acc, gs0, gs1):
    c = lax.axis_index("c")
    s = lax.axis_index("s")
    bufs = [(src0, dst0, rows0, gs0), (src1, dst1, rows1, gs1)]

    def load_and_gather(k, it, q):
        srcb, dstb, rowsb, gs = bufs[k]
        row = s * ROWS_PER_TILE + it * KJ
        pltpu.sync_copy(src_hbm.at[q, pl.ds(row, KJ)], srcb)
        pltpu.sync_copy(dst_hbm.at[pl.ds(row, KJ)], dstb)
        for j in range(KJ):
            pltpu.async_copy(xs_hbm.at[srcb.at[j]],
                             rowsb.at[pl.ds(j * G, G)], gs)

    def wait_gather(k):
        _, _, rowsb, gs = bufs[k]
        pltpu.make_async_copy(xs_hbm.at[pl.ds(0, CHUNK)], rowsb, gs).wait()

    def scatter_sync(k):
        _, dstb, rowsb, _ = bufs[k]
        for j in range(KJ):
            pltpu.sync_copy(rowsb.at[pl.ds(j * G, G)],
                            acc.at[dstb.at[j]], add=True)

    for p in range(NQ // NC):
        q = c * (NQ // NC) + p
        pltpu.sync_copy(zeros_hbm.at[pl.ds(s * STRIPE, STRIPE)],
                        acc.at[pl.ds(s * STRIPE, STRIPE)])
        plsc.subcore_barrier()

        load_and_gather(0, 0, q)

        def body(i2, carry):
            a = 2 * i2 + 1
            load_and_gather(1, a, q)
            wait_gather(0)
            scatter_sync(0)
            load_and_gather(0, a + 1, q)
            wait_gather(1)
            scatter_sync(1)
            return carry

        lax.fori_loop(0, (CH_PER_TILE - 1) // 2, body, 0)
        wait_gather(0)
        scatter_sync(0)
        plsc.subcore_barrier()
        pltpu.sync_copy(acc.at[pl.ds(s * STRIPE, STRIPE)],
                        out_hbm.at[pl.ds(s * STRIPE, STRIPE),
                                   pl.ds(q * QW, QW)])
        plsc.subcore_barrier()


_sc_scatter = functools.partial(
    pl.kernel,
    mesh=plsc.VectorSubcoreMesh(core_axis_name="c", subcore_axis_name="s"),
    compiler_params=pltpu.CompilerParams(use_tc_tiling_on_sc=False),
    out_type=jax.ShapeDtypeStruct((ACC_ROWS, H), jnp.float32),
    scratch_types=(
        [pltpu.VMEM((KJ, G), jnp.int32)] * 4
        + [pltpu.VMEM((CHUNK, QW), jnp.float32)] * 2
        + [pltpu.VMEM_SHARED((ACC_ROWS, QW), jnp.float32)]
        + [pltpu.SemaphoreType.DMA] * 2
    ),
)(_sc_scatter_body)


def _lin0_body(xp_ref, w_ref, b_ref, cnt_ref, x0_ref, dinv_ref, xs_ref):
    deg = cnt_ref[:, :1] + 1.0
    d = lax.rsqrt(deg)
    x0 = jnp.maximum(
        jnp.dot(xp_ref[...], w_ref[...], preferred_element_type=jnp.float32)
        + b_ref[...], 0.0)
    x0_ref[...] = x0
    dinv_ref[...] = d
    xs_ref[...] = d * x0


def _lin0(x_param, lin0_w, lin0_b, cnt):
    return pl.pallas_call(
        _lin0_body,
        grid=(NBLK,),
        in_specs=[
            pl.BlockSpec((BN, F_IN), lambda i: (i, 0)),
            pl.BlockSpec((F_IN, H), lambda i: (0, 0)),
            pl.BlockSpec((1, H), lambda i: (0, 0)),
            pl.BlockSpec((BN, H), lambda i: (i, 0)),
        ],
        out_specs=[
            pl.BlockSpec((BN, H), lambda i: (i, 0)),
            pl.BlockSpec((BN, 1), lambda i: (i, 0)),
            pl.BlockSpec((BN, H), lambda i: (i, 0)),
        ],
        out_shape=[
            jax.ShapeDtypeStruct((N, H), jnp.float32),
            jax.ShapeDtypeStruct((N, 1), jnp.float32),
            jax.ShapeDtypeStruct((N, H), jnp.float32),
        ],
    )(x_param, lin0_w, lin0_b, cnt)


def _layer_body(agg_ref, x_ref, x0_ref, dinv_ref, w_ref, xp_ref, xs_ref):
    aggc = agg_ref[...]
    d = dinv_ref[...]
    h = (1.0 - ALPHA) * (d * aggc + (d * d) * x_ref[...]) + ALPHA * x0_ref[...]
    xp = jnp.maximum(
        jnp.dot(h, w_ref[...], preferred_element_type=jnp.float32), 0.0)
    xp_ref[...] = xp
    xs_ref[...] = d * xp


def _layer(agg, x, x0, dinv, w):
    return pl.pallas_call(
        _layer_body,
        grid=(NBLK,),
        in_specs=[
            pl.BlockSpec((BN, H), lambda i: (i, 0)),
            pl.BlockSpec((BN, H), lambda i: (i, 0)),
            pl.BlockSpec((BN, H), lambda i: (i, 0)),
            pl.BlockSpec((BN, 1), lambda i: (i, 0)),
            pl.BlockSpec((H, H), lambda i: (0, 0)),
        ],
        out_specs=[
            pl.BlockSpec((BN, H), lambda i: (i, 0)),
            pl.BlockSpec((BN, H), lambda i: (i, 0)),
        ],
        out_shape=[
            jax.ShapeDtypeStruct((N, H), jnp.float32),
            jax.ShapeDtypeStruct((N, H), jnp.float32),
        ],
    )(agg, x, x0, dinv, w)


def _mlp_body(x_ref, w1_ref, b1_ref, w2_ref, b2_ref, w3_ref, b3_ref,
              ow_ref, ob_ref, out_ref):
    t = jnp.maximum(
        jnp.dot(x_ref[...], w1_ref[...], preferred_element_type=jnp.float32)
        + b1_ref[...], 0.0)
    t = jnp.maximum(
        jnp.dot(t, w2_ref[...], preferred_element_type=jnp.float32)
        + b2_ref[...], 0.0)
    t = jnp.dot(t, w3_ref[...], preferred_element_type=jnp.float32) + b3_ref[...]
    out_ref[...] = (
        jnp.dot(t, ow_ref[...], preferred_element_type=jnp.float32) + ob_ref[...])


def _mlp(x, w1, b1, w2, b2, w3, b3, ow, ob):
    return pl.pallas_call(
        _mlp_body,
        grid=(NBLK,),
        in_specs=[
            pl.BlockSpec((BN, H), lambda i: (i, 0)),
            pl.BlockSpec((H, H1P), lambda i: (0, 0)),
            pl.BlockSpec((1, H1P), lambda i: (0, 0)),
            pl.BlockSpec((H1P, H2P), lambda i: (0, 0)),
            pl.BlockSpec((1, H2P), lambda i: (0, 0)),
            pl.BlockSpec((H2P, R), lambda i: (0, 0)),
            pl.BlockSpec((1, R), lambda i: (0, 0)),
            pl.BlockSpec((R, C), lambda i: (0, 0)),
            pl.BlockSpec((1, C), lambda i: (0, 0)),
        ],
        out_specs=pl.BlockSpec((BN, C), lambda i: (i, 0)),
        out_shape=jax.ShapeDtypeStruct((N, C), jnp.float32),
    )(x, w1, b1, w2, b2, w3, b3, ow, ob)


def kernel(edge_index, x_param, lin0_w, lin0_b, conv_w, mlp_w1, mlp_b1,
           mlp_w2, mlp_b2, mlp_w3, mlp_b3, out_w, out_b):
    src = edge_index[0]
    dst = edge_index[1]
    pad = E_PAD - E
    srcp = jnp.concatenate([src, jnp.zeros((pad,), jnp.int32)])
    src4 = (NQ * srcp)[None, :] + jnp.arange(NQ, dtype=jnp.int32)[:, None]
    src4 = src4.reshape(NQ, EDGE_ROWS, G)
    dstp = jnp.concatenate([dst, jnp.full((pad,), N, jnp.int32)])
    dst2 = dstp.reshape(EDGE_ROWS, G)
    zeros = jnp.zeros((ACC_ROWS, QW), jnp.float32)
    ones_tbl = jnp.ones((NQ * N, QW), jnp.float32)

    betas = jnp.asarray([math.log(THETA / (i + 1) + 1.0) for i in range(L)],
                        jnp.float32)
    eye = jnp.eye(H, dtype=jnp.float32)
    wp = (1.0 - betas)[:, None, None] * eye + betas[:, None, None] * conv_w

    w1 = jnp.pad(mlp_w1, ((0, 0), (0, H1P - H1)))
    b1 = jnp.pad(mlp_b1, (0, H1P - H1)).reshape(1, H1P)
    w2 = jnp.pad(mlp_w2, ((0, H1P - H1), (0, H2P - H2)))
    b2 = jnp.pad(mlp_b2, (0, H2P - H2)).reshape(1, H2P)
    w3 = jnp.pad(mlp_w3, ((0, H2P - H2), (0, 0)))
    b3 = mlp_b3.reshape(1, R)
    ob = out_b.reshape(1, C)

    cnt = _sc_scatter(src4, dst2, ones_tbl, zeros)
    x0, dinv, xs = _lin0(x_param, lin0_w, lin0_b.reshape(1, H), cnt)
    x = x0
    for i in range(L):
        agg = _sc_scatter(src4, dst2, xs.reshape(NQ * N, QW), zeros)
        x, xs = _layer(agg, x, x0, dinv, wp[i])
    return _mlp(x, w1, b1, w2, b2, w3, b3, out_w, ob)
